# baseline (speedup 1.0000x reference)
import jax
import jax.numpy as jnp
from jax.experimental import pallas as pl
from jax.experimental.pallas import tpu as pltpu

N_TOK = 4096; DIM = 1024; HEADS = 16; DH = 64; BALL = 256
WIN = 16; BC = 16; SC = 8; BF = 16
NB = N_TOK // BALL
NW = (BALL - BC) // SC + 1
NG = NW + 1
HALF = DH // 2
NF = BALL // BF
SCALE = DH ** -0.5
F32 = jnp.float32
NEG = -1e10


def _prep_body(x_ref, pos_ref, pe_w_ref, pe_b_ref, rms_g_ref, w_qkv_ref,
               w_comb_ref, b_comb_ref, q_ref, k_ref, v_ref, gates_ref):
    posb = pos_ref[...]
    rel = posb - jnp.mean(posb, axis=0, keepdims=True)
    xb = x_ref[...] + jnp.dot(rel, pe_w_ref[...], preferred_element_type=F32) \
        + pe_b_ref[...]
    ms = jnp.mean(xb * xb, axis=-1, keepdims=True)
    xn = xb * jax.lax.rsqrt(ms + 1e-6) * rms_g_ref[...]
    qkv = jnp.dot(xn, w_qkv_ref[...], preferred_element_type=F32)
    gates_ref[...] = jax.nn.sigmoid(
        jnp.dot(xn, w_comb_ref[...], preferred_element_type=F32)
        + b_comb_ref[...])
    for h in range(HEADS):
        q_ref[0, h] = qkv[:, h * DH:(h + 1) * DH]
        k_ref[0, h] = qkv[:, DIM + h * DH:DIM + (h + 1) * DH]
        v_ref[0, h] = qkv[:, 2 * DIM + h * DH:2 * DIM + (h + 1) * DH]


def _swap(t):
    return pltpu.roll(t, HALF, axis=t.ndim - 1)


def _comp_body(k_ref, v_ref, cos_ref, sin_ref, kpe_ref, vpe_ref,
               mk_ref, mv_ref, kw1_ref, kw2_ref, vw1_ref, vw2_ref,
               ck_ref, cv_ref):
    def branch(t, pe, mem, w1, w2, out_ref):
        wins = [t[:, s0:s0 + BC, :] for s0 in range(0, SC * NW, SC)]
        tw = jnp.stack(wins, axis=1) + pe[None]
        flat = tw.reshape(NB * NW, BC * DH)
        h1 = jnp.maximum(jnp.dot(flat, w1[0], preferred_element_type=F32), 0.)
        c = jnp.dot(h1, w2[0], preferred_element_type=F32)
        out_ref[0, :, 1:NG, :] = c.reshape(NB, NW, DH)
        out_ref[0, :, 0:1, :] = jnp.broadcast_to(mem[None], (NB, 1, DH))

    kh = k_ref[:, 0]
    kr = kh * cos_ref[...][None] + _swap(kh) * sin_ref[...][None]
    branch(kr, kpe_ref[0], mk_ref[0], kw1_ref, kw2_ref, ck_ref)
    branch(v_ref[:, 0], vpe_ref[0], mv_ref[0], vw1_ref, vw2_ref, cv_ref)


def _attn_body(q_ref, k_ref, v_ref, ck_ref, cv_ref, gates_ref,
               cos_ref, sin_ref, cbias_ref, pool_ref, fidx_ref,
               jblk_ref, cab_ref, causb_ref, sbias_ref, wout_ref, o_ref):
    cos = cos_ref[...]
    sin = sin_ref[...]
    outs = []
    for h in range(HEADS):
        qh = q_ref[0, h]
        kh = k_ref[0, h]
        qr = qh * cos + _swap(qh) * sin
        kr = kh * cos + _swap(kh) * sin
        v = v_ref[0, h]
        g = gates_ref[0, h]

        csim = jax.lax.dot_general(qr, ck_ref[h, 0], (((1,), (1,)), ((), ())),
                                   preferred_element_type=F32) * SCALE \
            + cbias_ref[...]
        cmax = jnp.max(csim, axis=-1, keepdims=True)
        ce = jnp.exp(csim - cmax)
        cattn = ce / jnp.sum(ce, axis=-1, keepdims=True)
        c_out = jnp.dot(cattn, cv_ref[h, 0], preferred_element_type=F32)

        pooled = jnp.dot(cattn, pool_ref[...], preferred_element_type=F32)
        pmax = jnp.max(pooled, axis=-1, keepdims=True)
        sel = jnp.min(jnp.where(pooled == pmax, fidx_ref[...], float(NF)),
                      axis=-1, keepdims=True)

        sim = jax.lax.dot_general(qr, kr, (((1,), (1,)), ((), ())),
                                  preferred_element_type=F32) * SCALE
        fbias = jnp.maximum(cab_ref[...],
                            jnp.where(jblk_ref[...] == sel, 0.0, NEG)
                            + causb_ref[...])
        fsim = sim + fbias
        fmax = jnp.max(fsim, axis=-1, keepdims=True)
        fe = jnp.exp(fsim - fmax)
        fattn = fe * (1.0 / jnp.sum(fe, axis=-1, keepdims=True))
        f_out = jnp.dot(fattn, v, preferred_element_type=F32)

        ssim = sim + sbias_ref[...]
        smax = jnp.max(ssim, axis=-1, keepdims=True)
        se = jnp.exp(ssim - smax)
        sattn = se * (1.0 / jnp.sum(se, axis=-1, keepdims=True))
        s_out = jnp.dot(sattn, v, preferred_element_type=F32)

        outs.append(g[:, 0:1] * c_out + g[:, 1:2] * f_out + g[:, 2:3] * s_out)

    y = jnp.concatenate(outs, axis=1)
    o_ref[0] = jnp.dot(y, wout_ref[...], preferred_element_type=F32)


def kernel(x, pos, pe_w, pe_b, rms_g, w_qkv, k_posemb, v_posemb, k_w1, k_w2,
           v_w1, v_w2, mem_k, mem_v, w_comb, b_comb, w_out):
    freqs = 1.0 / (10000.0 ** (jnp.arange(HALF, dtype=F32) / HALF))
    ang = jnp.arange(BALL, dtype=F32)[:, None] * freqs[None, :]
    cosv, sinv = jnp.cos(ang), jnp.sin(ang)
    cosd = jnp.concatenate([cosv, cosv], axis=1)
    sind = jnp.concatenate([-sinv, sinv], axis=1)

    iar = jnp.arange(BALL)
    starts = jnp.arange(NW) * SC
    pool = jnp.concatenate(
        [jnp.zeros((1, NF), F32),
         jax.nn.one_hot(starts // BF, NF, dtype=F32)], axis=0)
    fidx = jnp.broadcast_to(jnp.arange(NF, dtype=F32)[None], (BALL, NF))
    cvis = jnp.concatenate(
        [jnp.ones((BALL, 1), bool),
         (starts + BC - 1)[None, :] < iar[:, None]], axis=1)
    cbias = jnp.where(cvis, 0.0, NEG).astype(F32)
    causal = iar[None, :] <= iar[:, None]
    jblk = jnp.broadcast_to((iar // BF)[None].astype(F32), (BALL, BALL))
    cab = jnp.where(causal & (iar[None, :] // BF == iar[:, None] // BF),
                    0.0, NEG).astype(F32)
    causb = jnp.where(causal, 0.0, NEG).astype(F32)
    diff = iar[:, None] - iar[None, :]
    sbias = jnp.where((diff >= 0) & (diff < WIN), 0.0, NEG).astype(F32)

    hshape = jax.ShapeDtypeStruct((NB, HEADS, BALL, DH), F32)
    hblock = pl.BlockSpec((1, HEADS, BALL, DH), lambda b: (b, 0, 0, 0))

    q, k, v, gates = pl.pallas_call(
        _prep_body,
        grid=(NB,),
        in_specs=[
            pl.BlockSpec((BALL, DIM), lambda b: (b, 0)),
            pl.BlockSpec((BALL, 3), lambda b: (b, 0)),
            pl.BlockSpec((3, DIM), lambda b: (0, 0)),
            pl.BlockSpec((1, DIM), lambda b: (0, 0)),
            pl.BlockSpec((1, DIM), lambda b: (0, 0)),
            pl.BlockSpec((DIM, 3 * DIM), lambda b: (0, 0)),
            pl.BlockSpec((DIM, 3 * HEADS), lambda b: (0, 0)),
            pl.BlockSpec((1, 3 * HEADS), lambda b: (0, 0)),
        ],
        out_specs=[hblock, hblock, hblock,
                   pl.BlockSpec((BALL, 3 * HEADS), lambda b: (b, 0))],
        out_shape=[hshape, hshape, hshape,
                   jax.ShapeDtypeStruct((N_TOK, 3 * HEADS), F32)],
    )(x, pos, pe_w, pe_b.reshape(1, DIM), rms_g.reshape(1, DIM), w_qkv,
      w_comb, b_comb.reshape(1, 3 * HEADS))

    gates = gates.reshape(NB, BALL, HEADS, 3).transpose(0, 2, 1, 3)

    ckf, cvf = pl.pallas_call(
        _comp_body,
        grid=(HEADS,),
        in_specs=[
            pl.BlockSpec((NB, 1, BALL, DH), lambda h: (0, h, 0, 0)),
            pl.BlockSpec((NB, 1, BALL, DH), lambda h: (0, h, 0, 0)),
            pl.BlockSpec((BALL, DH), lambda h: (0, 0)),
            pl.BlockSpec((BALL, DH), lambda h: (0, 0)),
            pl.BlockSpec((1, BC, DH), lambda h: (h, 0, 0)),
            pl.BlockSpec((1, BC, DH), lambda h: (h, 0, 0)),
            pl.BlockSpec((1, 1, DH), lambda h: (h, 0, 0)),
            pl.BlockSpec((1, 1, DH), lambda h: (h, 0, 0)),
            pl.BlockSpec((1, BC * DH, BC * DH), lambda h: (h, 0, 0)),
            pl.BlockSpec((1, BC * DH, DH), lambda h: (h, 0, 0)),
            pl.BlockSpec((1, BC * DH, BC * DH), lambda h: (h, 0, 0)),
            pl.BlockSpec((1, BC * DH, DH), lambda h: (h, 0, 0)),
        ],
        out_specs=[
            pl.BlockSpec((1, NB, NG, DH), lambda h: (h, 0, 0, 0)),
            pl.BlockSpec((1, NB, NG, DH), lambda h: (h, 0, 0, 0)),
        ],
        out_shape=[
            jax.ShapeDtypeStruct((HEADS, NB, NG, DH), F32),
            jax.ShapeDtypeStruct((HEADS, NB, NG, DH), F32),
        ],
    )(k, v, cosd, sind, k_posemb, v_posemb, mem_k, mem_v,
      k_w1, k_w2, v_w1, v_w2)

    out = pl.pallas_call(
        _attn_body,
        grid=(NB,),
        in_specs=[
            hblock, hblock, hblock,
            pl.BlockSpec((HEADS, 1, NG, DH), lambda b: (0, b, 0, 0)),
            pl.BlockSpec((HEADS, 1, NG, DH), lambda b: (0, b, 0, 0)),
            pl.BlockSpec((1, HEADS, BALL, 3), lambda b: (b, 0, 0, 0)),
            pl.BlockSpec((BALL, DH), lambda b: (0, 0)),
            pl.BlockSpec((BALL, DH), lambda b: (0, 0)),
            pl.BlockSpec((BALL, NG), lambda b: (0, 0)),
            pl.BlockSpec((NG, NF), lambda b: (0, 0)),
            pl.BlockSpec((BALL, NF), lambda b: (0, 0)),
            pl.BlockSpec((BALL, BALL), lambda b: (0, 0)),
            pl.BlockSpec((BALL, BALL), lambda b: (0, 0)),
            pl.BlockSpec((BALL, BALL), lambda b: (0, 0)),
            pl.BlockSpec((BALL, BALL), lambda b: (0, 0)),
            pl.BlockSpec((DIM, DIM), lambda b: (0, 0)),
        ],
        out_specs=pl.BlockSpec((1, BALL, DIM), lambda b: (b, 0, 0)),
        out_shape=jax.ShapeDtypeStruct((NB, BALL, DIM), F32),
    )(q, k, v, ckf, cvf, gates, cosd, sind,
      cbias, pool, fidx, jblk, cab, causb, sbias, w_out)
    return out.reshape(N_TOK, DIM)


# speedup vs baseline: 1.1326x; 1.1326x over previous
import jax
import jax.numpy as jnp
from jax.experimental import pallas as pl
from jax.experimental.pallas import tpu as pltpu

N_TOK = 4096; DIM = 1024; HEADS = 16; DH = 64; BALL = 256
WIN = 16; BC = 16; SC = 8; BF = 16
NB = N_TOK // BALL
NW = (BALL - BC) // SC + 1
NG = NW + 1
HALF = DH // 2
NF = BALL // BF
SCALE = DH ** -0.5
F32 = jnp.float32
NEG = -1e10


def _prep_body(x_ref, pos_ref, pe_w_ref, pe_b_ref, rms_g_ref, w_qkv_ref,
               w_comb_ref, b_comb_ref, q_ref, qs_ref, k_ref, ks_ref, v_ref,
               gates_ref):
    posb = pos_ref[...]
    rel = posb - jnp.mean(posb, axis=0, keepdims=True)
    xb = x_ref[...] + jnp.dot(rel, pe_w_ref[...], preferred_element_type=F32) \
        + pe_b_ref[...]
    ms = jnp.mean(xb * xb, axis=-1, keepdims=True)
    xn = xb * jax.lax.rsqrt(ms + 1e-6) * rms_g_ref[...]
    qkv = jnp.dot(xn, w_qkv_ref[...], preferred_element_type=F32)
    gates_ref[...] = jax.nn.sigmoid(
        jnp.dot(xn, w_comb_ref[...], preferred_element_type=F32)
        + b_comb_ref[...])
    for h in range(HEADS):
        oq, ok, ov = h * DH, DIM + h * DH, 2 * DIM + h * DH
        q_ref[0, h] = qkv[:, oq:oq + DH]
        k_ref[0, h] = qkv[:, ok:ok + DH]
        v_ref[0, h] = qkv[:, ov:ov + DH]
        qs_ref[0, h, :, 0:HALF] = qkv[:, oq + HALF:oq + DH]
        qs_ref[0, h, :, HALF:DH] = qkv[:, oq:oq + HALF]
        ks_ref[0, h, :, 0:HALF] = qkv[:, ok + HALF:ok + DH]
        ks_ref[0, h, :, HALF:DH] = qkv[:, ok:ok + HALF]


def _comp_body(k_ref, ks_ref, v_ref, cos_ref, sin_ref, kpe_ref, vpe_ref,
               mk_ref, mv_ref, kw1_ref, kw2_ref, vw1_ref, vw2_ref,
               ck_ref, cv_ref):
    def branch(t, pe, mem, w1, w2, out_ref):
        wins = [t[:, s0:s0 + BC, :] for s0 in range(0, SC * NW, SC)]
        tw = jnp.stack(wins, axis=1) + pe[None]
        flat = tw.reshape(NB * NW, BC * DH)
        h1 = jnp.maximum(jnp.dot(flat, w1[0], preferred_element_type=F32), 0.)
        c = jnp.dot(h1, w2[0], preferred_element_type=F32)
        out_ref[0, :, 1:NG, :] = c.reshape(NB, NW, DH)
        out_ref[0, :, 0:1, :] = jnp.broadcast_to(mem[None], (NB, 1, DH))

    kr = k_ref[:, 0] * cos_ref[...][None] + ks_ref[:, 0] * sin_ref[...][None]
    branch(kr, kpe_ref[0], mk_ref[0], kw1_ref, kw2_ref, ck_ref)
    branch(v_ref[:, 0], vpe_ref[0], mv_ref[0], vw1_ref, vw2_ref, cv_ref)


def _attn_body(q_ref, qs_ref, k_ref, ks_ref, v_ref, ck_ref, cv_ref,
               gates_ref, cos_ref, sin_ref, cbias_ref, pool_ref, fidx_ref,
               jblk_ref, cab_ref, causb_ref, sbias_ref, o_ref):
    cos = cos_ref[...]
    sin = sin_ref[...]
    for h in range(HEADS):
        qr = q_ref[0, h] * cos + qs_ref[0, h] * sin
        kr = k_ref[0, h] * cos + ks_ref[0, h] * sin
        v = v_ref[0, h]
        g = gates_ref[0, h]

        csim = jax.lax.dot_general(qr, ck_ref[h, 0], (((1,), (1,)), ((), ())),
                                   preferred_element_type=F32) * SCALE \
            + cbias_ref[...]
        cmax = jnp.max(csim, axis=-1, keepdims=True)
        ce = jnp.exp(csim - cmax)
        cattn = ce / jnp.sum(ce, axis=-1, keepdims=True)
        c_out = jnp.dot(cattn, cv_ref[h, 0], preferred_element_type=F32)

        pooled = jnp.dot(cattn, pool_ref[...], preferred_element_type=F32)
        pmax = jnp.max(pooled, axis=-1, keepdims=True)
        sel = jnp.min(jnp.where(pooled == pmax, fidx_ref[...], float(NF)),
                      axis=-1, keepdims=True)

        sim = jax.lax.dot_general(qr, kr, (((1,), (1,)), ((), ())),
                                  preferred_element_type=F32) * SCALE
        fbias = jnp.maximum(cab_ref[...],
                            jnp.where(jblk_ref[...] == sel, 0.0, NEG)
                            + causb_ref[...])
        fsim = sim + fbias
        fmax = jnp.max(fsim, axis=-1, keepdims=True)
        fe = jnp.exp(fsim - fmax)
        fattn = fe * (1.0 / jnp.sum(fe, axis=-1, keepdims=True))
        f_out = jnp.dot(fattn, v, preferred_element_type=F32)

        ssim = sim + sbias_ref[...]
        smax = jnp.max(ssim, axis=-1, keepdims=True)
        se = jnp.exp(ssim - smax)
        sattn = se * (1.0 / jnp.sum(se, axis=-1, keepdims=True))
        s_out = jnp.dot(sattn, v, preferred_element_type=F32)

        o_ref[0, :, h * DH:(h + 1) * DH] = \
            g[:, 0:1] * c_out + g[:, 1:2] * f_out + g[:, 2:3] * s_out


def _proj_body(y_ref, w_ref, o_ref):
    o_ref[...] = jnp.dot(y_ref[...], w_ref[...], preferred_element_type=F32)


def kernel(x, pos, pe_w, pe_b, rms_g, w_qkv, k_posemb, v_posemb, k_w1, k_w2,
           v_w1, v_w2, mem_k, mem_v, w_comb, b_comb, w_out):
    freqs = 1.0 / (10000.0 ** (jnp.arange(HALF, dtype=F32) / HALF))
    ang = jnp.arange(BALL, dtype=F32)[:, None] * freqs[None, :]
    cosv, sinv = jnp.cos(ang), jnp.sin(ang)
    cosd = jnp.concatenate([cosv, cosv], axis=1)
    sind = jnp.concatenate([-sinv, sinv], axis=1)

    iar = jnp.arange(BALL)
    starts = jnp.arange(NW) * SC
    pool = jnp.concatenate(
        [jnp.zeros((1, NF), F32),
         jax.nn.one_hot(starts // BF, NF, dtype=F32)], axis=0)
    fidx = jnp.broadcast_to(jnp.arange(NF, dtype=F32)[None], (BALL, NF))
    cvis = jnp.concatenate(
        [jnp.ones((BALL, 1), bool),
         (starts + BC - 1)[None, :] < iar[:, None]], axis=1)
    cbias = jnp.where(cvis, 0.0, NEG).astype(F32)
    causal = iar[None, :] <= iar[:, None]
    jblk = jnp.broadcast_to((iar // BF)[None].astype(F32), (BALL, BALL))
    cab = jnp.where(causal & (iar[None, :] // BF == iar[:, None] // BF),
                    0.0, NEG).astype(F32)
    causb = jnp.where(causal, 0.0, NEG).astype(F32)
    diff = iar[:, None] - iar[None, :]
    sbias = jnp.where((diff >= 0) & (diff < WIN), 0.0, NEG).astype(F32)

    hshape = jax.ShapeDtypeStruct((NB, HEADS, BALL, DH), F32)
    hblock = pl.BlockSpec((1, HEADS, BALL, DH), lambda b: (b, 0, 0, 0))

    q, qs, k, ks, v, gates = pl.pallas_call(
        _prep_body,
        grid=(NB,),
        in_specs=[
            pl.BlockSpec((BALL, DIM), lambda b: (b, 0)),
            pl.BlockSpec((BALL, 3), lambda b: (b, 0)),
            pl.BlockSpec((3, DIM), lambda b: (0, 0)),
            pl.BlockSpec((1, DIM), lambda b: (0, 0)),
            pl.BlockSpec((1, DIM), lambda b: (0, 0)),
            pl.BlockSpec((DIM, 3 * DIM), lambda b: (0, 0)),
            pl.BlockSpec((DIM, 3 * HEADS), lambda b: (0, 0)),
            pl.BlockSpec((1, 3 * HEADS), lambda b: (0, 0)),
        ],
        out_specs=[hblock, hblock, hblock, hblock, hblock,
                   pl.BlockSpec((BALL, 3 * HEADS), lambda b: (b, 0))],
        out_shape=[hshape, hshape, hshape, hshape, hshape,
                   jax.ShapeDtypeStruct((N_TOK, 3 * HEADS), F32)],
    )(x, pos, pe_w, pe_b.reshape(1, DIM), rms_g.reshape(1, DIM), w_qkv,
      w_comb, b_comb.reshape(1, 3 * HEADS))

    gates = gates.reshape(NB, BALL, HEADS, 3).transpose(0, 2, 1, 3)

    ckf, cvf = pl.pallas_call(
        _comp_body,
        grid=(HEADS,),
        in_specs=[
            pl.BlockSpec((NB, 1, BALL, DH), lambda h: (0, h, 0, 0)),
            pl.BlockSpec((NB, 1, BALL, DH), lambda h: (0, h, 0, 0)),
            pl.BlockSpec((NB, 1, BALL, DH), lambda h: (0, h, 0, 0)),
            pl.BlockSpec((BALL, DH), lambda h: (0, 0)),
            pl.BlockSpec((BALL, DH), lambda h: (0, 0)),
            pl.BlockSpec((1, BC, DH), lambda h: (h, 0, 0)),
            pl.BlockSpec((1, BC, DH), lambda h: (h, 0, 0)),
            pl.BlockSpec((1, 1, DH), lambda h: (h, 0, 0)),
            pl.BlockSpec((1, 1, DH), lambda h: (h, 0, 0)),
            pl.BlockSpec((1, BC * DH, BC * DH), lambda h: (h, 0, 0)),
            pl.BlockSpec((1, BC * DH, DH), lambda h: (h, 0, 0)),
            pl.BlockSpec((1, BC * DH, BC * DH), lambda h: (h, 0, 0)),
            pl.BlockSpec((1, BC * DH, DH), lambda h: (h, 0, 0)),
        ],
        out_specs=[
            pl.BlockSpec((1, NB, NG, DH), lambda h: (h, 0, 0, 0)),
            pl.BlockSpec((1, NB, NG, DH), lambda h: (h, 0, 0, 0)),
        ],
        out_shape=[
            jax.ShapeDtypeStruct((HEADS, NB, NG, DH), F32),
            jax.ShapeDtypeStruct((HEADS, NB, NG, DH), F32),
        ],
    )(k, ks, v, cosd, sind, k_posemb, v_posemb, mem_k, mem_v,
      k_w1, k_w2, v_w1, v_w2)

    y = pl.pallas_call(
        _attn_body,
        grid=(NB,),
        in_specs=[
            hblock, hblock, hblock, hblock, hblock,
            pl.BlockSpec((HEADS, 1, NG, DH), lambda b: (0, b, 0, 0)),
            pl.BlockSpec((HEADS, 1, NG, DH), lambda b: (0, b, 0, 0)),
            pl.BlockSpec((1, HEADS, BALL, 3), lambda b: (b, 0, 0, 0)),
            pl.BlockSpec((BALL, DH), lambda b: (0, 0)),
            pl.BlockSpec((BALL, DH), lambda b: (0, 0)),
            pl.BlockSpec((BALL, NG), lambda b: (0, 0)),
            pl.BlockSpec((NG, NF), lambda b: (0, 0)),
            pl.BlockSpec((BALL, NF), lambda b: (0, 0)),
            pl.BlockSpec((BALL, BALL), lambda b: (0, 0)),
            pl.BlockSpec((BALL, BALL), lambda b: (0, 0)),
            pl.BlockSpec((BALL, BALL), lambda b: (0, 0)),
            pl.BlockSpec((BALL, BALL), lambda b: (0, 0)),
        ],
        out_specs=pl.BlockSpec((1, BALL, DIM), lambda b: (b, 0, 0)),
        out_shape=jax.ShapeDtypeStruct((NB, BALL, DIM), F32),
    )(q, qs, k, ks, v, ckf, cvf, gates, cosd, sind,
      cbias, pool, fidx, jblk, cab, causb, sbias)

    out = pl.pallas_call(
        _proj_body,
        grid=(8,),
        in_specs=[
            pl.BlockSpec((2, BALL, DIM), lambda i: (i, 0, 0)),
            pl.BlockSpec((DIM, DIM), lambda i: (0, 0)),
        ],
        out_specs=pl.BlockSpec((2, BALL, DIM), lambda i: (i, 0, 0)),
        out_shape=jax.ShapeDtypeStruct((NB, BALL, DIM), F32),
    )(y, w_out)
    return out.reshape(N_TOK, DIM)


# speedup vs baseline: 1.1616x; 1.0256x over previous
import jax
import jax.numpy as jnp
from jax.experimental import pallas as pl
from jax.experimental.pallas import tpu as pltpu

N_TOK = 4096; DIM = 1024; HEADS = 16; DH = 64; BALL = 256
WIN = 16; BC = 16; SC = 8; BF = 16
NB = N_TOK // BALL
NW = (BALL - BC) // SC + 1
NG = NW + 1
HALF = DH // 2
NF = BALL // BF
SCALE = DH ** -0.5
F32 = jnp.float32
NEG = -1e10
BPP = 2


def _prep_body(x_ref, pos_ref, pe_w_ref, pe_b_ref, rms_g_ref, w_qkv_ref,
               w_comb_ref, b_comb_ref, q_ref, qs_ref, k_ref, ks_ref, v_ref,
               gates_ref):
    posb = pos_ref[...]
    rel = posb - jnp.mean(posb, axis=0, keepdims=True)
    xb = x_ref[...] + jnp.dot(rel, pe_w_ref[...], preferred_element_type=F32) \
        + pe_b_ref[...]
    ms = jnp.mean(xb * xb, axis=-1, keepdims=True)
    xn = xb * jax.lax.rsqrt(ms + 1e-6) * rms_g_ref[...]
    qkv = jnp.dot(xn, w_qkv_ref[...], preferred_element_type=F32)
    gates_ref[...] = jax.nn.sigmoid(
        jnp.dot(xn, w_comb_ref[...], preferred_element_type=F32)
        + b_comb_ref[...])
    for h in range(HEADS):
        oq, ok, ov = h * DH, DIM + h * DH, 2 * DIM + h * DH
        q_ref[0, h] = qkv[:, oq:oq + DH]
        k_ref[0, h] = qkv[:, ok:ok + DH]
        v_ref[0, h] = qkv[:, ov:ov + DH]
        qs_ref[0, h, :, 0:HALF] = qkv[:, oq + HALF:oq + DH]
        qs_ref[0, h, :, HALF:DH] = qkv[:, oq:oq + HALF]
        ks_ref[0, h, :, 0:HALF] = qkv[:, ok + HALF:ok + DH]
        ks_ref[0, h, :, HALF:DH] = qkv[:, ok:ok + HALF]


def _comp_body(k_ref, ks_ref, v_ref, cos_ref, sin_ref, kpe_ref, vpe_ref,
               mk_ref, mv_ref, kw1_ref, kw2_ref, vw1_ref, vw2_ref,
               ck_ref, cv_ref):
    def branch(t, pe, mem, w1, w2, out_ref):
        wins = [t[:, s0:s0 + BC, :] for s0 in range(0, SC * NW, SC)]
        tw = jnp.stack(wins, axis=1) + pe[None]
        flat = tw.reshape(NB * NW, BC * DH)
        h1 = jnp.maximum(jnp.dot(flat, w1[0], preferred_element_type=F32), 0.)
        c = jnp.dot(h1, w2[0], preferred_element_type=F32)
        out_ref[0, :, 1:NG, :] = c.reshape(NB, NW, DH)
        out_ref[0, :, 0:1, :] = jnp.broadcast_to(mem[None], (NB, 1, DH))

    kr = k_ref[:, 0] * cos_ref[...][None] + ks_ref[:, 0] * sin_ref[...][None]
    branch(kr, kpe_ref[0], mk_ref[0], kw1_ref, kw2_ref, ck_ref)
    branch(v_ref[:, 0], vpe_ref[0], mv_ref[0], vw1_ref, vw2_ref, cv_ref)


def _attn_body(q_ref, qs_ref, k_ref, ks_ref, v_ref, ck_ref, cv_ref,
               gates_ref, cos_ref, sin_ref, cbias_ref, pool_ref, fidx_ref,
               jblk_ref, cab_ref, causb_ref, sbias_ref, o_ref):
    cos = cos_ref[...]
    sin = sin_ref[...]
    for bi in range(BPP):
      for h in range(HEADS):
        qr = q_ref[bi, h] * cos + qs_ref[bi, h] * sin
        kr = k_ref[bi, h] * cos + ks_ref[bi, h] * sin
        v = v_ref[bi, h]
        g = gates_ref[bi * BALL:(bi + 1) * BALL, 3 * h:3 * h + 3]

        csim = jax.lax.dot_general(qr, ck_ref[h, bi], (((1,), (1,)), ((), ())),
                                   preferred_element_type=F32) * SCALE \
            + cbias_ref[...]
        cmax = jnp.max(csim, axis=-1, keepdims=True)
        ce = jnp.exp(csim - cmax)
        cattn = ce / jnp.sum(ce, axis=-1, keepdims=True)
        c_out = jnp.dot(cattn, cv_ref[h, bi], preferred_element_type=F32)

        pooled = jnp.dot(cattn, pool_ref[...], preferred_element_type=F32)
        pmax = jnp.max(pooled, axis=-1, keepdims=True)
        sel = jnp.min(jnp.where(pooled == pmax, fidx_ref[...], float(NF)),
                      axis=-1, keepdims=True)

        sim = jax.lax.dot_general(qr, kr, (((1,), (1,)), ((), ())),
                                  preferred_element_type=F32) * SCALE
        fbias = jnp.maximum(cab_ref[...],
                            jnp.where(jblk_ref[...] == sel, 0.0, NEG)
                            + causb_ref[...])
        fsim = sim + fbias
        fmax = jnp.max(fsim, axis=-1, keepdims=True)
        fe = jnp.exp(fsim - fmax)
        fattn = fe * (1.0 / jnp.sum(fe, axis=-1, keepdims=True))
        f_out = jnp.dot(fattn, v, preferred_element_type=F32)

        ssim = sim + sbias_ref[...]
        smax = jnp.max(ssim, axis=-1, keepdims=True)
        se = jnp.exp(ssim - smax)
        sattn = se * (1.0 / jnp.sum(se, axis=-1, keepdims=True))
        s_out = jnp.dot(sattn, v, preferred_element_type=F32)

        o_ref[bi, :, h * DH:(h + 1) * DH] = \
            g[:, 0:1] * c_out + g[:, 1:2] * f_out + g[:, 2:3] * s_out


def _proj_body(y_ref, w_ref, o_ref):
    o_ref[...] = jnp.dot(y_ref[...], w_ref[...], preferred_element_type=F32)


def kernel(x, pos, pe_w, pe_b, rms_g, w_qkv, k_posemb, v_posemb, k_w1, k_w2,
           v_w1, v_w2, mem_k, mem_v, w_comb, b_comb, w_out):
    freqs = 1.0 / (10000.0 ** (jnp.arange(HALF, dtype=F32) / HALF))
    ang = jnp.arange(BALL, dtype=F32)[:, None] * freqs[None, :]
    cosv, sinv = jnp.cos(ang), jnp.sin(ang)
    cosd = jnp.concatenate([cosv, cosv], axis=1)
    sind = jnp.concatenate([-sinv, sinv], axis=1)

    iar = jnp.arange(BALL)
    starts = jnp.arange(NW) * SC
    pool = jnp.concatenate(
        [jnp.zeros((1, NF), F32),
         jax.nn.one_hot(starts // BF, NF, dtype=F32)], axis=0)
    fidx = jnp.broadcast_to(jnp.arange(NF, dtype=F32)[None], (BALL, NF))
    cvis = jnp.concatenate(
        [jnp.ones((BALL, 1), bool),
         (starts + BC - 1)[None, :] < iar[:, None]], axis=1)
    cbias = jnp.where(cvis, 0.0, NEG).astype(F32)
    causal = iar[None, :] <= iar[:, None]
    jblk = jnp.broadcast_to((iar // BF)[None].astype(F32), (BALL, BALL))
    cab = jnp.where(causal & (iar[None, :] // BF == iar[:, None] // BF),
                    0.0, NEG).astype(F32)
    causb = jnp.where(causal, 0.0, NEG).astype(F32)
    diff = iar[:, None] - iar[None, :]
    sbias = jnp.where((diff >= 0) & (diff < WIN), 0.0, NEG).astype(F32)

    hshape = jax.ShapeDtypeStruct((NB, HEADS, BALL, DH), F32)
    hblock = pl.BlockSpec((1, HEADS, BALL, DH), lambda b: (b, 0, 0, 0))

    q, qs, k, ks, v, gates = pl.pallas_call(
        _prep_body,
        grid=(NB,),
        in_specs=[
            pl.BlockSpec((BALL, DIM), lambda b: (b, 0)),
            pl.BlockSpec((BALL, 3), lambda b: (b, 0)),
            pl.BlockSpec((3, DIM), lambda b: (0, 0)),
            pl.BlockSpec((1, DIM), lambda b: (0, 0)),
            pl.BlockSpec((1, DIM), lambda b: (0, 0)),
            pl.BlockSpec((DIM, 3 * DIM), lambda b: (0, 0)),
            pl.BlockSpec((DIM, 3 * HEADS), lambda b: (0, 0)),
            pl.BlockSpec((1, 3 * HEADS), lambda b: (0, 0)),
        ],
        out_specs=[hblock, hblock, hblock, hblock, hblock,
                   pl.BlockSpec((BALL, 3 * HEADS), lambda b: (b, 0))],
        out_shape=[hshape, hshape, hshape, hshape, hshape,
                   jax.ShapeDtypeStruct((N_TOK, 3 * HEADS), F32)],
    )(x, pos, pe_w, pe_b.reshape(1, DIM), rms_g.reshape(1, DIM), w_qkv,
      w_comb, b_comb.reshape(1, 3 * HEADS))

    ckf, cvf = pl.pallas_call(
        _comp_body,
        grid=(HEADS,),
        in_specs=[
            pl.BlockSpec((NB, 1, BALL, DH), lambda h: (0, h, 0, 0)),
            pl.BlockSpec((NB, 1, BALL, DH), lambda h: (0, h, 0, 0)),
            pl.BlockSpec((NB, 1, BALL, DH), lambda h: (0, h, 0, 0)),
            pl.BlockSpec((BALL, DH), lambda h: (0, 0)),
            pl.BlockSpec((BALL, DH), lambda h: (0, 0)),
            pl.BlockSpec((1, BC, DH), lambda h: (h, 0, 0)),
            pl.BlockSpec((1, BC, DH), lambda h: (h, 0, 0)),
            pl.BlockSpec((1, 1, DH), lambda h: (h, 0, 0)),
            pl.BlockSpec((1, 1, DH), lambda h: (h, 0, 0)),
            pl.BlockSpec((1, BC * DH, BC * DH), lambda h: (h, 0, 0)),
            pl.BlockSpec((1, BC * DH, DH), lambda h: (h, 0, 0)),
            pl.BlockSpec((1, BC * DH, BC * DH), lambda h: (h, 0, 0)),
            pl.BlockSpec((1, BC * DH, DH), lambda h: (h, 0, 0)),
        ],
        out_specs=[
            pl.BlockSpec((1, NB, NG, DH), lambda h: (h, 0, 0, 0)),
            pl.BlockSpec((1, NB, NG, DH), lambda h: (h, 0, 0, 0)),
        ],
        out_shape=[
            jax.ShapeDtypeStruct((HEADS, NB, NG, DH), F32),
            jax.ShapeDtypeStruct((HEADS, NB, NG, DH), F32),
        ],
    )(k, ks, v, cosd, sind, k_posemb, v_posemb, mem_k, mem_v,
      k_w1, k_w2, v_w1, v_w2)

    ablock = pl.BlockSpec((BPP, HEADS, BALL, DH), lambda b: (b, 0, 0, 0))
    y = pl.pallas_call(
        _attn_body,
        grid=(NB // BPP,),
        in_specs=[
            ablock, ablock, ablock, ablock, ablock,
            pl.BlockSpec((HEADS, BPP, NG, DH), lambda b: (0, b, 0, 0)),
            pl.BlockSpec((HEADS, BPP, NG, DH), lambda b: (0, b, 0, 0)),
            pl.BlockSpec((BPP * BALL, 3 * HEADS), lambda b: (b, 0)),
            pl.BlockSpec((BALL, DH), lambda b: (0, 0)),
            pl.BlockSpec((BALL, DH), lambda b: (0, 0)),
            pl.BlockSpec((BALL, NG), lambda b: (0, 0)),
            pl.BlockSpec((NG, NF), lambda b: (0, 0)),
            pl.BlockSpec((BALL, NF), lambda b: (0, 0)),
            pl.BlockSpec((BALL, BALL), lambda b: (0, 0)),
            pl.BlockSpec((BALL, BALL), lambda b: (0, 0)),
            pl.BlockSpec((BALL, BALL), lambda b: (0, 0)),
            pl.BlockSpec((BALL, BALL), lambda b: (0, 0)),
        ],
        out_specs=pl.BlockSpec((BPP, BALL, DIM), lambda b: (b, 0, 0)),
        out_shape=jax.ShapeDtypeStruct((NB, BALL, DIM), F32),
    )(q, qs, k, ks, v, ckf, cvf, gates, cosd, sind,
      cbias, pool, fidx, jblk, cab, causb, sbias)

    out = pl.pallas_call(
        _proj_body,
        grid=(8,),
        in_specs=[
            pl.BlockSpec((2, BALL, DIM), lambda i: (i, 0, 0)),
            pl.BlockSpec((DIM, DIM), lambda i: (0, 0)),
        ],
        out_specs=pl.BlockSpec((2, BALL, DIM), lambda i: (i, 0, 0)),
        out_shape=jax.ShapeDtypeStruct((NB, BALL, DIM), F32),
    )(y, w_out)
    return out.reshape(N_TOK, DIM)
